# baseline (speedup 1.0000x reference)
import jax
import jax.numpy as jnp
from jax import lax
from jax.experimental import pallas as pl
from jax.experimental.pallas import tpu as pltpu
from jax.experimental.pallas import tpu_sc as plsc

B = 16384
EMB = 32
NC = 2
NS = 16
NW = NC * NS
B_PER_W = B // NW


def _sc_gather_kernel(ug_t, ig_t, um_t, im_t, u_hbm, i_hbm, out_o,
                      uidx_v, iidx_v, buf, sem):
    wid = lax.axis_index("s") * NC + lax.axis_index("c")
    base = wid * B_PER_W
    pltpu.sync_copy(u_hbm.at[pl.ds(base, B_PER_W)], uidx_v)
    pltpu.sync_copy(i_hbm.at[pl.ds(base, B_PER_W)], iidx_v)

    @pl.loop(0, B_PER_W // 16)
    def _(kk):
        ua = uidx_v[pl.ds(kk * 16, 16)]
        ia = iidx_v[pl.ds(kk * 16, 16)]
        for l in range(16):
            j = kk * 16 + l
            r = ua[l]
            pltpu.async_copy(ug_t.at[r], buf.at[j, pl.ds(0, EMB)], sem)
            pltpu.async_copy(um_t.at[r], buf.at[j, pl.ds(2 * EMB, EMB)], sem)
            r2 = ia[l]
            pltpu.async_copy(ig_t.at[r2], buf.at[j, pl.ds(EMB, EMB)], sem)
            pltpu.async_copy(im_t.at[r2], buf.at[j, pl.ds(3 * EMB, EMB)], sem)

    pltpu.make_async_copy(out_o.at[pl.ds(base, B_PER_W)], buf, sem).wait()
    pltpu.sync_copy(buf, out_o.at[pl.ds(base, B_PER_W)])


def _sc_gather(u_idx, i_idx, U_gmf, I_gmf, U_mlp, I_mlp):
    mesh = plsc.VectorSubcoreMesh(core_axis_name="c", subcore_axis_name="s")
    k = pl.kernel(
        _sc_gather_kernel,
        mesh=mesh,
        out_type=jax.ShapeDtypeStruct((B, 4 * EMB), jnp.float32),
        scratch_types=[
            pltpu.VMEM((B_PER_W,), jnp.int32),
            pltpu.VMEM((B_PER_W,), jnp.int32),
            pltpu.VMEM((B_PER_W, 4 * EMB), jnp.float32),
            pltpu.SemaphoreType.DMA,
        ],
    )
    return k(U_gmf, I_gmf, U_mlp, I_mlp, u_idx, i_idx)


def _tc_dense_kernel(x, w1, b1, w2, b2, w3, b3, out):
    xv = x[...]
    mul = xv[:, 0:EMB] * xv[:, EMB:2 * EMB]
    w1v = w1[...]
    h = jnp.dot(xv[:, 2 * EMB:3 * EMB], w1v[:EMB, :],
                preferred_element_type=jnp.float32)
    h += jnp.dot(xv[:, 3 * EMB:4 * EMB], w1v[EMB:, :],
                 preferred_element_type=jnp.float32)
    h = jnp.maximum(h + b1[...], 0.0)
    h2 = jnp.dot(h, w2[...], preferred_element_type=jnp.float32)
    h2 = jnp.maximum(h2 + b2[...], 0.0)
    w3v = w3[...]
    o = jnp.dot(mul, w3v[:EMB, :], preferred_element_type=jnp.float32)
    o += jnp.dot(h2, w3v[EMB:, :], preferred_element_type=jnp.float32)
    out[...] = o + b3[...]


def _tc_dense(packed, W1, b1, W2, b2, W3, b3):
    Bb = 2048
    grid = (B // Bb,)

    def full(shape):
        return pl.BlockSpec(shape, lambda i: tuple(0 for _ in shape))

    return pl.pallas_call(
        _tc_dense_kernel,
        grid=grid,
        in_specs=[
            pl.BlockSpec((Bb, 4 * EMB), lambda i: (i, 0)),
            full(W1.shape), full(b1.shape),
            full(W2.shape), full(b2.shape),
            full(W3.shape), full(b3.shape),
        ],
        out_specs=pl.BlockSpec((Bb, 1), lambda i: (i, 0)),
        out_shape=jax.ShapeDtypeStruct((B, 1), jnp.float32),
    )(packed, W1, b1, W2, b2, W3, b3)


def kernel(user, item, U_gmf, I_gmf, U_mlp, I_mlp, W1, b1, W2, b2, W3, b3):
    u_idx = user.reshape(B)
    i_idx = item.reshape(B)
    packed = _sc_gather(u_idx, i_idx, U_gmf, I_gmf, U_mlp, I_mlp)
    return _tc_dense(packed,
                     W1, b1.reshape(1, 64), W2, b2.reshape(1, 32),
                     W3, b3.reshape(1, 1))


# speedup vs baseline: 2.1391x; 2.1391x over previous
import dataclasses

import jax
import jax.numpy as jnp
from jax import lax
from jax.experimental import pallas as pl
from jax.experimental.pallas import tpu as pltpu
from jax.experimental.pallas import tpu_sc as plsc

B = 16384
EMB = 32
NC = 2
NS = 16
NW = NC * NS
B_PER_W = B // NW
LANES = 128


def _issue_fetch(tabs, fb, slot, ru, ri, sem):
    ug_t, ig_t, um_t, im_t = tabs
    ou = pl.multiple_of(ru - lax.rem(ru, LANES), LANES)
    oi = pl.multiple_of(ri - lax.rem(ri, LANES), LANES)
    pltpu.async_copy(ug_t.at[:, pl.ds(ou, LANES)],
                     fb.at[slot, pl.ds(0, EMB), :], sem)
    pltpu.async_copy(ig_t.at[:, pl.ds(oi, LANES)],
                     fb.at[slot, pl.ds(EMB, EMB), :], sem)
    pltpu.async_copy(um_t.at[:, pl.ds(ou, LANES)],
                     fb.at[slot, pl.ds(2 * EMB, EMB), :], sem)
    pltpu.async_copy(im_t.at[:, pl.ds(oi, LANES)],
                     fb.at[slot, pl.ds(3 * EMB, EMB), :], sem)


def _select(fb, slot, buf, j, ru, ri):
    rmu = lax.rem(ru, LANES)
    rmi = lax.rem(ri, LANES)
    iota = lax.iota(jnp.int32, 16)
    for t, rm in ((0, rmu), (1, rmi), (2, rmu), (3, rmi)):
        lane = jnp.full((16,), rm, jnp.int32)
        for g in range(2):
            dim_idx = iota + (t * EMB + g * 16)
            v = plsc.load_gather(fb.at[slot], [dim_idx, lane])
            buf[j, pl.ds(t * EMB + g * 16, 16)] = v


def _sc_gather_kernel(ug_t, ig_t, um_t, im_t, u_hbm, i_hbm, out_o,
                      uidx_v, iidx_v, buf, fb, sem0, sem1):
    wid = lax.axis_index("s") * NC + lax.axis_index("c")
    base = wid * B_PER_W
    pltpu.sync_copy(u_hbm.at[pl.ds(base, B_PER_W)], uidx_v)
    pltpu.sync_copy(i_hbm.at[pl.ds(base, B_PER_W)], iidx_v)
    tabs = (ug_t, ig_t, um_t, im_t)
    sems = (sem0, sem1)

    @pl.loop(0, B_PER_W // 16)
    def _(kk):
        ua = uidx_v[pl.ds(kk * 16, 16)]
        ia = iidx_v[pl.ds(kk * 16, 16)]
        _issue_fetch(tabs, fb, 0, ua[0], ia[0], sems[0])
        for l in range(16):
            j = kk * 16 + l
            if l < 15:
                _issue_fetch(tabs, fb, (l + 1) % 2, ua[l + 1], ia[l + 1],
                             sems[(l + 1) % 2])
            pltpu.make_async_copy(out_o.at[pl.ds(0, 4 * EMB)],
                                  fb.at[l % 2], sems[l % 2]).wait()
            _select(fb, l % 2, buf, j, ua[l], ia[l])

    pltpu.sync_copy(buf, out_o.at[pl.ds(base, B_PER_W)])


def _sc_gather(u_idx, i_idx, U_gmf_T, I_gmf_T, U_mlp_T, I_mlp_T):
    mesh = plsc.VectorSubcoreMesh(core_axis_name="c", subcore_axis_name="s")
    cp = pltpu.CompilerParams()
    if "needs_layout_passes" in pltpu.CompilerParams.__dataclass_fields__:
        cp = dataclasses.replace(cp, needs_layout_passes=False)
    k = pl.kernel(
        _sc_gather_kernel,
        mesh=mesh,
        compiler_params=cp,
        out_type=jax.ShapeDtypeStruct((B, 4 * EMB), jnp.float32),
        scratch_types=[
            pltpu.VMEM((B_PER_W,), jnp.int32),
            pltpu.VMEM((B_PER_W,), jnp.int32),
            pltpu.VMEM((B_PER_W, 4 * EMB), jnp.float32),
            pltpu.VMEM((2, 4 * EMB, LANES), jnp.float32),
            pltpu.SemaphoreType.DMA,
            pltpu.SemaphoreType.DMA,
        ],
    )
    return k(U_gmf_T, I_gmf_T, U_mlp_T, I_mlp_T, u_idx, i_idx)


def _tc_dense_kernel(x, w1, b1, w2, b2, w3, b3, out):
    xv = x[...]
    mul = xv[:, 0:EMB] * xv[:, EMB:2 * EMB]
    w1v = w1[...]
    h = jnp.dot(xv[:, 2 * EMB:3 * EMB], w1v[:EMB, :],
                preferred_element_type=jnp.float32)
    h += jnp.dot(xv[:, 3 * EMB:4 * EMB], w1v[EMB:, :],
                 preferred_element_type=jnp.float32)
    h = jnp.maximum(h + b1[...], 0.0)
    h2 = jnp.dot(h, w2[...], preferred_element_type=jnp.float32)
    h2 = jnp.maximum(h2 + b2[...], 0.0)
    w3v = w3[...]
    o = jnp.dot(mul, w3v[:EMB, :], preferred_element_type=jnp.float32)
    o += jnp.dot(h2, w3v[EMB:, :], preferred_element_type=jnp.float32)
    out[...] = o + b3[...]


def _tc_dense(packed, W1, b1, W2, b2, W3, b3):
    Bb = 2048
    grid = (B // Bb,)

    def full(shape):
        return pl.BlockSpec(shape, lambda i: tuple(0 for _ in shape))

    return pl.pallas_call(
        _tc_dense_kernel,
        grid=grid,
        in_specs=[
            pl.BlockSpec((Bb, 4 * EMB), lambda i: (i, 0)),
            full(W1.shape), full(b1.shape),
            full(W2.shape), full(b2.shape),
            full(W3.shape), full(b3.shape),
        ],
        out_specs=pl.BlockSpec((Bb, 1), lambda i: (i, 0)),
        out_shape=jax.ShapeDtypeStruct((B, 1), jnp.float32),
    )(packed, W1, b1, W2, b2, W3, b3)


def kernel(user, item, U_gmf, I_gmf, U_mlp, I_mlp, W1, b1, W2, b2, W3, b3):
    u_idx = user.reshape(B)
    i_idx = item.reshape(B)
    packed = _sc_gather(u_idx, i_idx, U_gmf.T, I_gmf.T, U_mlp.T, I_mlp.T)
    return _tc_dense(packed,
                     W1, b1.reshape(1, 64), W2, b2.reshape(1, 32),
                     W3, b3.reshape(1, 1))


# speedup vs baseline: 2.6464x; 1.2372x over previous
import dataclasses

import jax
import jax.numpy as jnp
from jax import lax
from jax.experimental import pallas as pl
from jax.experimental.pallas import tpu as pltpu
from jax.experimental.pallas import tpu_sc as plsc

B = 16384
EMB = 32
NC = 2
NS = 16
NW = NC * NS
B_PER_W = B // NW
LANES = 128


def _issue_fetch(tabs, fb, slot, ru, ri, sem):
    ug_t, ig_t, um_t, im_t = tabs
    ou = pl.multiple_of(ru - lax.rem(ru, LANES), LANES)
    oi = pl.multiple_of(ri - lax.rem(ri, LANES), LANES)
    pltpu.async_copy(ug_t.at[:, pl.ds(ou, LANES)],
                     fb.at[slot, pl.ds(0, EMB), :], sem)
    pltpu.async_copy(ig_t.at[:, pl.ds(oi, LANES)],
                     fb.at[slot, pl.ds(EMB, EMB), :], sem)
    pltpu.async_copy(um_t.at[:, pl.ds(ou, LANES)],
                     fb.at[slot, pl.ds(2 * EMB, EMB), :], sem)
    pltpu.async_copy(im_t.at[:, pl.ds(oi, LANES)],
                     fb.at[slot, pl.ds(3 * EMB, EMB), :], sem)


def _select(fb, slot, buf, j, ru, ri):
    rmu = lax.rem(ru, LANES)
    rmi = lax.rem(ri, LANES)
    iota = lax.iota(jnp.int32, 16)
    for t, rm in ((0, rmu), (1, rmi), (2, rmu), (3, rmi)):
        lane = jnp.full((16,), rm, jnp.int32)
        for g in range(2):
            dim_idx = iota + (t * EMB + g * 16)
            v = plsc.load_gather(fb.at[slot], [dim_idx, lane])
            buf[j, pl.ds(t * EMB + g * 16, 16)] = v


DEPTH = 4
HALF = B_PER_W // 2


def _sc_gather_kernel(ug_t, ig_t, um_t, im_t, u_hbm, i_hbm, out_o,
                      uidx_v, iidx_v, buf, fb, sem0, sem1, sem2, sem3):
    wid = lax.axis_index("s") * NC + lax.axis_index("c")
    base = wid * B_PER_W
    pltpu.sync_copy(u_hbm.at[pl.ds(base, B_PER_W)], uidx_v)
    pltpu.sync_copy(i_hbm.at[pl.ds(base, B_PER_W)], iidx_v)
    tabs = (ug_t, ig_t, um_t, im_t)
    sems = (sem0, sem1, sem2, sem3)

    ua0 = uidx_v[pl.ds(0, 16)]
    ia0 = iidx_v[pl.ds(0, 16)]
    for l in range(DEPTH - 1):
        _issue_fetch(tabs, fb, l, ua0[l], ia0[l], sems[l])

    @pl.loop(0, B_PER_W // 16)
    def _(kk):
        ua = uidx_v[pl.ds(kk * 16, 16)]
        ia = iidx_v[pl.ds(kk * 16, 16)]
        nxt = jnp.minimum((kk + 1) * 16, B_PER_W - 16)
        ua_n = uidx_v[pl.ds(nxt, 16)]
        ia_n = iidx_v[pl.ds(nxt, 16)]
        for l in range(16):
            j = kk * 16 + l
            la = l + DEPTH - 1
            ru_a = ua[la] if la < 16 else ua_n[la - 16]
            ri_a = ia[la] if la < 16 else ia_n[la - 16]
            slot_a = la % DEPTH

            @pl.when(jnp.logical_or(kk < B_PER_W // 16 - 1, la < 16))
            def _():
                _issue_fetch(tabs, fb, slot_a, ru_a, ri_a, sems[slot_a])

            pltpu.make_async_copy(out_o.at[pl.ds(0, 4 * EMB)],
                                  fb.at[l % DEPTH], sems[l % DEPTH]).wait()
            _select(fb, l % DEPTH, buf, (kk % 16) * 16 + l, ua[l], ia[l])

        @pl.when(kk == B_PER_W // 32 - 1)
        def _():
            pltpu.sync_copy(buf, out_o.at[pl.ds(base, HALF)])

        @pl.when(kk == B_PER_W // 16 - 1)
        def _():
            pltpu.sync_copy(buf, out_o.at[pl.ds(base + HALF, HALF)])


def _sc_gather(u_idx, i_idx, U_gmf_T, I_gmf_T, U_mlp_T, I_mlp_T):
    mesh = plsc.VectorSubcoreMesh(core_axis_name="c", subcore_axis_name="s")
    cp = pltpu.CompilerParams()
    if "needs_layout_passes" in pltpu.CompilerParams.__dataclass_fields__:
        cp = dataclasses.replace(cp, needs_layout_passes=False)
    k = pl.kernel(
        _sc_gather_kernel,
        mesh=mesh,
        compiler_params=cp,
        out_type=jax.ShapeDtypeStruct((B, 4 * EMB), jnp.float32),
        scratch_types=[
            pltpu.VMEM((B_PER_W,), jnp.int32),
            pltpu.VMEM((B_PER_W,), jnp.int32),
            pltpu.VMEM((HALF, 4 * EMB), jnp.float32),
            pltpu.VMEM((DEPTH, 4 * EMB, LANES), jnp.float32),
            pltpu.SemaphoreType.DMA,
            pltpu.SemaphoreType.DMA,
            pltpu.SemaphoreType.DMA,
            pltpu.SemaphoreType.DMA,
        ],
    )
    return k(U_gmf_T, I_gmf_T, U_mlp_T, I_mlp_T, u_idx, i_idx)


def _tc_dense_kernel(x, w1, b1, w2, b2, w3, b3, out):
    xv = x[...]
    mul = xv[:, 0:EMB] * xv[:, EMB:2 * EMB]
    w1v = w1[...]
    h = jnp.dot(xv[:, 2 * EMB:3 * EMB], w1v[:EMB, :],
                preferred_element_type=jnp.float32)
    h += jnp.dot(xv[:, 3 * EMB:4 * EMB], w1v[EMB:, :],
                 preferred_element_type=jnp.float32)
    h = jnp.maximum(h + b1[...], 0.0)
    h2 = jnp.dot(h, w2[...], preferred_element_type=jnp.float32)
    h2 = jnp.maximum(h2 + b2[...], 0.0)
    w3v = w3[...]
    o = jnp.dot(mul, w3v[:EMB, :], preferred_element_type=jnp.float32)
    o += jnp.dot(h2, w3v[EMB:, :], preferred_element_type=jnp.float32)
    out[...] = o + b3[...]


def _tc_dense(packed, W1, b1, W2, b2, W3, b3):
    Bb = 2048
    grid = (B // Bb,)

    def full(shape):
        return pl.BlockSpec(shape, lambda i: tuple(0 for _ in shape))

    return pl.pallas_call(
        _tc_dense_kernel,
        grid=grid,
        in_specs=[
            pl.BlockSpec((Bb, 4 * EMB), lambda i: (i, 0)),
            full(W1.shape), full(b1.shape),
            full(W2.shape), full(b2.shape),
            full(W3.shape), full(b3.shape),
        ],
        out_specs=pl.BlockSpec((Bb, 1), lambda i: (i, 0)),
        out_shape=jax.ShapeDtypeStruct((B, 1), jnp.float32),
    )(packed, W1, b1, W2, b2, W3, b3)


def kernel(user, item, U_gmf, I_gmf, U_mlp, I_mlp, W1, b1, W2, b2, W3, b3):
    u_idx = user.reshape(B)
    i_idx = item.reshape(B)
    packed = _sc_gather(u_idx, i_idx, U_gmf.T, I_gmf.T, U_mlp.T, I_mlp.T)
    return _tc_dense(packed,
                     W1, b1.reshape(1, 64), W2, b2.reshape(1, 32),
                     W3, b3.reshape(1, 1))


# speedup vs baseline: 2.9172x; 1.1023x over previous
import dataclasses

import jax
import jax.numpy as jnp
from jax import lax
from jax.experimental import pallas as pl
from jax.experimental.pallas import tpu as pltpu
from jax.experimental.pallas import tpu_sc as plsc

B = 16384
EMB = 32
NC = 2
NS = 16
NW = NC * NS
B_PER_W = B // NW
LANES = 128


def _issue_fetch(tabs, fb, slot, ru, ri, sem):
    ug_t, ig_t, um_t, im_t = tabs
    ou = pl.multiple_of(ru - lax.rem(ru, LANES), LANES)
    oi = pl.multiple_of(ri - lax.rem(ri, LANES), LANES)
    pltpu.async_copy(ug_t.at[:, pl.ds(ou, LANES)],
                     fb.at[slot, pl.ds(0, EMB), :], sem)
    pltpu.async_copy(ig_t.at[:, pl.ds(oi, LANES)],
                     fb.at[slot, pl.ds(EMB, EMB), :], sem)
    pltpu.async_copy(um_t.at[:, pl.ds(ou, LANES)],
                     fb.at[slot, pl.ds(2 * EMB, EMB), :], sem)
    pltpu.async_copy(im_t.at[:, pl.ds(oi, LANES)],
                     fb.at[slot, pl.ds(3 * EMB, EMB), :], sem)


def _select(fb, slot, buf, j, ru, ri):
    rmu = lax.rem(ru, LANES)
    rmi = lax.rem(ri, LANES)
    iota = lax.iota(jnp.int32, 16)
    for t, rm in ((0, rmu), (1, rmi), (2, rmu), (3, rmi)):
        lane = jnp.full((16,), rm, jnp.int32)
        for g in range(2):
            dim_idx = iota + (t * EMB + g * 16)
            v = plsc.load_gather(fb.at[slot], [dim_idx, lane])
            buf[j, pl.ds(t * EMB + g * 16, 16)] = v


DEPTH = 6
CHUNK = 128


def _idx_at(idx_v, j):
    jv = jnp.full((16,), j, jnp.int32)
    return plsc.load_gather(idx_v, [jv])[0]


def _sc_gather_kernel(ug_t, ig_t, um_t, im_t, u_hbm, i_hbm, out_o,
                      uidx_v, iidx_v, buf, fb, sems):
    wid = lax.axis_index("s") * NC + lax.axis_index("c")
    base = wid * B_PER_W
    pltpu.sync_copy(u_hbm.at[pl.ds(base, B_PER_W)], uidx_v)
    pltpu.sync_copy(i_hbm.at[pl.ds(base, B_PER_W)], iidx_v)
    tabs = (ug_t, ig_t, um_t, im_t)

    for l in range(DEPTH - 1):
        _issue_fetch(tabs, fb, l, _idx_at(uidx_v, l), _idx_at(iidx_v, l),
                     sems.at[l])

    @pl.loop(0, B_PER_W)
    def _(j):
        ja = j + DEPTH - 1
        slot_a = lax.rem(ja, DEPTH)

        @pl.when(ja < B_PER_W)
        def _():
            _issue_fetch(tabs, fb, slot_a, _idx_at(uidx_v, ja),
                         _idx_at(iidx_v, ja), sems.at[slot_a])

        slot = lax.rem(j, DEPTH)
        pltpu.make_async_copy(out_o.at[pl.ds(0, 4 * EMB)],
                              fb.at[slot], sems.at[slot]).wait()
        _select(fb, slot, buf, lax.rem(j, CHUNK),
                _idx_at(uidx_v, j), _idx_at(iidx_v, j))

        @pl.when(lax.rem(j, CHUNK) == CHUNK - 1)
        def _():
            dst = pl.multiple_of(base + (j - (CHUNK - 1)), CHUNK)
            pltpu.sync_copy(buf, out_o.at[pl.ds(dst, CHUNK)])


def _sc_gather(u_idx, i_idx, U_gmf_T, I_gmf_T, U_mlp_T, I_mlp_T):
    mesh = plsc.VectorSubcoreMesh(core_axis_name="c", subcore_axis_name="s")
    cp = pltpu.CompilerParams()
    if "needs_layout_passes" in pltpu.CompilerParams.__dataclass_fields__:
        cp = dataclasses.replace(cp, needs_layout_passes=False)
    k = pl.kernel(
        _sc_gather_kernel,
        mesh=mesh,
        compiler_params=cp,
        out_type=jax.ShapeDtypeStruct((B, 4 * EMB), jnp.float32),
        scratch_types=[
            pltpu.VMEM((B_PER_W,), jnp.int32),
            pltpu.VMEM((B_PER_W,), jnp.int32),
            pltpu.VMEM((CHUNK, 4 * EMB), jnp.float32),
            pltpu.VMEM((DEPTH, 4 * EMB, LANES), jnp.float32),
            pltpu.SemaphoreType.DMA((DEPTH,)),
        ],
    )
    return k(U_gmf_T, I_gmf_T, U_mlp_T, I_mlp_T, u_idx, i_idx)


def _tc_dense_kernel(x, w1, b1, w2, b2, w3, b3, out):
    xv = x[...]
    mul = xv[:, 0:EMB] * xv[:, EMB:2 * EMB]
    w1v = w1[...]
    h = jnp.dot(xv[:, 2 * EMB:3 * EMB], w1v[:EMB, :],
                preferred_element_type=jnp.float32)
    h += jnp.dot(xv[:, 3 * EMB:4 * EMB], w1v[EMB:, :],
                 preferred_element_type=jnp.float32)
    h = jnp.maximum(h + b1[...], 0.0)
    h2 = jnp.dot(h, w2[...], preferred_element_type=jnp.float32)
    h2 = jnp.maximum(h2 + b2[...], 0.0)
    w3v = w3[...]
    o = jnp.dot(mul, w3v[:EMB, :], preferred_element_type=jnp.float32)
    o += jnp.dot(h2, w3v[EMB:, :], preferred_element_type=jnp.float32)
    out[...] = o + b3[...]


def _tc_dense(packed, W1, b1, W2, b2, W3, b3):
    Bb = 2048
    grid = (B // Bb,)

    def full(shape):
        return pl.BlockSpec(shape, lambda i: tuple(0 for _ in shape))

    return pl.pallas_call(
        _tc_dense_kernel,
        grid=grid,
        in_specs=[
            pl.BlockSpec((Bb, 4 * EMB), lambda i: (i, 0)),
            full(W1.shape), full(b1.shape),
            full(W2.shape), full(b2.shape),
            full(W3.shape), full(b3.shape),
        ],
        out_specs=pl.BlockSpec((Bb, 1), lambda i: (i, 0)),
        out_shape=jax.ShapeDtypeStruct((B, 1), jnp.float32),
    )(packed, W1, b1, W2, b2, W3, b3)


def kernel(user, item, U_gmf, I_gmf, U_mlp, I_mlp, W1, b1, W2, b2, W3, b3):
    u_idx = user.reshape(B)
    i_idx = item.reshape(B)
    packed = _sc_gather(u_idx, i_idx, U_gmf.T, I_gmf.T, U_mlp.T, I_mlp.T)
    return _tc_dense(packed,
                     W1, b1.reshape(1, 64), W2, b2.reshape(1, 32),
                     W3, b3.reshape(1, 1))


# speedup vs baseline: 2.9297x; 1.0043x over previous
import dataclasses

import jax
import jax.numpy as jnp
from jax import lax
from jax.experimental import pallas as pl
from jax.experimental.pallas import tpu as pltpu
from jax.experimental.pallas import tpu_sc as plsc

B = 16384
EMB = 32
NC = 2
NS = 16
NW = NC * NS
B_PER_W = B // NW
LANES = 128


def _issue_fetch(tabs, fb, slot, ru, ri, sem):
    ug_t, ig_t, um_t, im_t = tabs
    ou = pl.multiple_of(ru - lax.rem(ru, LANES), LANES)
    oi = pl.multiple_of(ri - lax.rem(ri, LANES), LANES)
    pltpu.async_copy(ug_t.at[:, pl.ds(ou, LANES)],
                     fb.at[slot, pl.ds(0, EMB), :], sem)
    pltpu.async_copy(ig_t.at[:, pl.ds(oi, LANES)],
                     fb.at[slot, pl.ds(EMB, EMB), :], sem)
    pltpu.async_copy(um_t.at[:, pl.ds(ou, LANES)],
                     fb.at[slot, pl.ds(2 * EMB, EMB), :], sem)
    pltpu.async_copy(im_t.at[:, pl.ds(oi, LANES)],
                     fb.at[slot, pl.ds(3 * EMB, EMB), :], sem)


def _select(fb, slot, buf, j, ru, ri):
    rmu = lax.rem(ru, LANES)
    rmi = lax.rem(ri, LANES)
    iota = lax.iota(jnp.int32, 16)
    for t, rm in ((0, rmu), (1, rmi), (2, rmu), (3, rmi)):
        lane = jnp.full((16,), rm, jnp.int32)
        for g in range(2):
            dim_idx = iota + (t * EMB + g * 16)
            v = plsc.load_gather(fb.at[slot], [dim_idx, lane])
            buf[j, pl.ds(t * EMB + g * 16, 16)] = v


DEPTH = 7
CHUNK = 64


def _idx_at(idx_v, j):
    jv = jnp.full((16,), j, jnp.int32)
    return plsc.load_gather(idx_v, [jv])[0]


def _sc_gather_kernel(ug_t, ig_t, um_t, im_t, u_hbm, i_hbm, out_o,
                      uidx_v, iidx_v, buf, fb, sems):
    wid = lax.axis_index("s") * NC + lax.axis_index("c")
    base = wid * B_PER_W
    pltpu.sync_copy(u_hbm.at[pl.ds(base, B_PER_W)], uidx_v)
    pltpu.sync_copy(i_hbm.at[pl.ds(base, B_PER_W)], iidx_v)
    tabs = (ug_t, ig_t, um_t, im_t)

    for l in range(DEPTH - 1):
        _issue_fetch(tabs, fb, l, _idx_at(uidx_v, l), _idx_at(iidx_v, l),
                     sems.at[l])

    @pl.loop(0, B_PER_W)
    def _(j):
        ja = j + DEPTH - 1
        slot_a = lax.rem(ja, DEPTH)

        @pl.when(ja < B_PER_W)
        def _():
            _issue_fetch(tabs, fb, slot_a, _idx_at(uidx_v, ja),
                         _idx_at(iidx_v, ja), sems.at[slot_a])

        slot = lax.rem(j, DEPTH)
        pltpu.make_async_copy(out_o.at[pl.ds(0, 4 * EMB)],
                              fb.at[slot], sems.at[slot]).wait()
        _select(fb, slot, buf, lax.rem(j, CHUNK),
                _idx_at(uidx_v, j), _idx_at(iidx_v, j))

        @pl.when(lax.rem(j, CHUNK) == CHUNK - 1)
        def _():
            dst = pl.multiple_of(base + (j - (CHUNK - 1)), CHUNK)
            pltpu.sync_copy(buf, out_o.at[pl.ds(dst, CHUNK)])


def _sc_gather(u_idx, i_idx, U_gmf_T, I_gmf_T, U_mlp_T, I_mlp_T):
    mesh = plsc.VectorSubcoreMesh(core_axis_name="c", subcore_axis_name="s")
    cp = pltpu.CompilerParams()
    if "needs_layout_passes" in pltpu.CompilerParams.__dataclass_fields__:
        cp = dataclasses.replace(cp, needs_layout_passes=False)
    k = pl.kernel(
        _sc_gather_kernel,
        mesh=mesh,
        compiler_params=cp,
        out_type=jax.ShapeDtypeStruct((B, 4 * EMB), jnp.float32),
        scratch_types=[
            pltpu.VMEM((B_PER_W,), jnp.int32),
            pltpu.VMEM((B_PER_W,), jnp.int32),
            pltpu.VMEM((CHUNK, 4 * EMB), jnp.float32),
            pltpu.VMEM((DEPTH, 4 * EMB, LANES), jnp.float32),
            pltpu.SemaphoreType.DMA((DEPTH,)),
        ],
    )
    return k(U_gmf_T, I_gmf_T, U_mlp_T, I_mlp_T, u_idx, i_idx)


def _tc_dense_kernel(x, w1, b1, w2, b2, w3, b3, out):
    xv = x[...]
    mul = xv[:, 0:EMB] * xv[:, EMB:2 * EMB]
    w1v = w1[...]
    h = jnp.dot(xv[:, 2 * EMB:3 * EMB], w1v[:EMB, :],
                preferred_element_type=jnp.float32)
    h += jnp.dot(xv[:, 3 * EMB:4 * EMB], w1v[EMB:, :],
                 preferred_element_type=jnp.float32)
    h = jnp.maximum(h + b1[...], 0.0)
    h2 = jnp.dot(h, w2[...], preferred_element_type=jnp.float32)
    h2 = jnp.maximum(h2 + b2[...], 0.0)
    w3v = w3[...]
    o = jnp.dot(mul, w3v[:EMB, :], preferred_element_type=jnp.float32)
    o += jnp.dot(h2, w3v[EMB:, :], preferred_element_type=jnp.float32)
    out[...] = o + b3[...]


def _tc_dense(packed, W1, b1, W2, b2, W3, b3):
    Bb = 2048
    grid = (B // Bb,)

    def full(shape):
        return pl.BlockSpec(shape, lambda i: tuple(0 for _ in shape))

    return pl.pallas_call(
        _tc_dense_kernel,
        grid=grid,
        in_specs=[
            pl.BlockSpec((Bb, 4 * EMB), lambda i: (i, 0)),
            full(W1.shape), full(b1.shape),
            full(W2.shape), full(b2.shape),
            full(W3.shape), full(b3.shape),
        ],
        out_specs=pl.BlockSpec((Bb, 1), lambda i: (i, 0)),
        out_shape=jax.ShapeDtypeStruct((B, 1), jnp.float32),
    )(packed, W1, b1, W2, b2, W3, b3)


def kernel(user, item, U_gmf, I_gmf, U_mlp, I_mlp, W1, b1, W2, b2, W3, b3):
    u_idx = user.reshape(B)
    i_idx = item.reshape(B)
    packed = _sc_gather(u_idx, i_idx, U_gmf.T, I_gmf.T, U_mlp.T, I_mlp.T)
    return _tc_dense(packed,
                     W1, b1.reshape(1, 64), W2, b2.reshape(1, 32),
                     W3, b3.reshape(1, 1))
